# baseline (speedup 1.0000x reference)
import functools

import jax
import jax.numpy as jnp
from jax import lax
from jax.experimental import pallas as pl
from jax.experimental.pallas import tpu as pltpu
from jax.experimental.pallas import tpu_sc as plsc

NUM_OBJECTS = 1000000
EMBED_DIM = 64
BATCH = 16384

_info = plsc.get_sparse_core_info()
_NC, _NS, _L = _info.num_cores, _info.num_subcores, _info.num_lanes
_NW = _NC * _NS
_B_PER_W = BATCH // _NW
_VECS_PER_ROW = EMBED_DIM // _L


def _rsqrt16(s):
    bits = lax.bitcast_convert_type(s, jnp.int32)
    y = lax.bitcast_convert_type(jnp.int32(0x5F3759DF) - (bits >> 1),
                                 jnp.float32)
    half = s * 0.5
    for _ in range(3):
        y = y * (1.5 - half * y * y)
    return y


def _lane_shuffle(v, idx):
    dnums = lax.GatherDimensionNumbers(
        offset_dims=(), collapsed_slice_dims=(0,), start_index_map=(0,))
    return lax.gather(v, idx[:, None], dnums, slice_sizes=(1,),
                      mode=lax.GatherScatterMode.PROMISE_IN_BOUNDS)


def _sc_body(table_hbm, idx_hbm, out_hbm, idx_v, rows_v, sem):
    wid = lax.axis_index("s") * _NC + lax.axis_index("c")
    base = wid * _B_PER_W
    pltpu.sync_copy(idx_hbm.at[pl.ds(base, _B_PER_W)], idx_v)
    pltpu.async_copy(table_hbm.at[idx_v], rows_v, sem).wait()

    def row_body(i, _):
        v0 = rows_v[i, pl.ds(0 * _L, _L)]
        v1 = rows_v[i, pl.ds(1 * _L, _L)]
        v2 = rows_v[i, pl.ds(2 * _L, _L)]
        v3 = rows_v[i, pl.ds(3 * _L, _L)]
        ss = v0 * v0 + v1 * v1 + v2 * v2 + v3 * v3
        lane = lax.iota(jnp.int32, _L)
        for k in (8, 4, 2, 1):
            ss = ss + _lane_shuffle(ss, lane ^ k)
        s = jnp.maximum(ss, 1e-12)
        inv = _rsqrt16(s)
        rows_v[i, pl.ds(0 * _L, _L)] = v0 * inv
        rows_v[i, pl.ds(1 * _L, _L)] = v1 * inv
        rows_v[i, pl.ds(2 * _L, _L)] = v2 * inv
        rows_v[i, pl.ds(3 * _L, _L)] = v3 * inv
        return 0

    lax.fori_loop(0, _B_PER_W, row_body, 0)
    pltpu.sync_copy(rows_v, out_hbm.at[pl.ds(base, _B_PER_W)])


@jax.jit
def _encode(ids, table):
    mesh = plsc.VectorSubcoreMesh(core_axis_name="c", subcore_axis_name="s")
    call = pl.kernel(
        _sc_body,
        mesh=mesh,
        out_type=jax.ShapeDtypeStruct((BATCH, EMBED_DIM), jnp.float32),
        scratch_types=[
            pltpu.VMEM((_B_PER_W,), jnp.int32),
            pltpu.VMEM((_B_PER_W, EMBED_DIM), jnp.float32),
            pltpu.SemaphoreType.DMA,
        ],
        compiler_params=pltpu.CompilerParams(use_tc_tiling_on_sc=False),
    )
    return call(table, ids.astype(jnp.int32))


def kernel(ids, table):
    return _encode(ids, table)


# speedup vs baseline: 1.6952x; 1.6952x over previous
import functools

import jax
import jax.numpy as jnp
from jax import lax
from jax.experimental import pallas as pl
from jax.experimental.pallas import tpu as pltpu
from jax.experimental.pallas import tpu_sc as plsc

NUM_OBJECTS = 1000000
EMBED_DIM = 64
BATCH = 16384

_info = plsc.get_sparse_core_info()
_NC, _NS, _L = _info.num_cores, _info.num_subcores, _info.num_lanes
_NW = _NC * _NS
_B_PER_W = BATCH // _NW
_VECS_PER_ROW = EMBED_DIM // _L


def _rsqrt16(s):
    bits = lax.bitcast_convert_type(s, jnp.int32)
    y = lax.bitcast_convert_type(jnp.int32(0x5F3759DF) - (bits >> 1),
                                 jnp.float32)
    half = s * 0.5
    for _ in range(3):
        y = y * (1.5 - half * y * y)
    return y


def _lane_shuffle(v, idx):
    dnums = lax.GatherDimensionNumbers(
        offset_dims=(), collapsed_slice_dims=(0,), start_index_map=(0,))
    return lax.gather(v, idx[:, None], dnums, slice_sizes=(1,),
                      mode=lax.GatherScatterMode.PROMISE_IN_BOUNDS)


def _sc_body(table_hbm, idx_hbm, out_hbm, idx_v, rows_v, sem):
    wid = lax.axis_index("s") * _NC + lax.axis_index("c")
    base = wid * _B_PER_W
    pltpu.sync_copy(idx_hbm.at[pl.ds(base, _B_PER_W)], idx_v)

    def gather_body(g, _):
        vec = idx_v[pl.ds(g * _L, _L)]
        for j in range(_L):
            r = vec[j]
            pltpu.async_copy(table_hbm.at[pl.ds(r, 1)],
                             rows_v.at[pl.ds(g * _L + j, 1)], sem)
        return 0

    lax.fori_loop(0, _B_PER_W // _L, gather_body, 0)
    pltpu.make_async_copy(table_hbm.at[pl.ds(0, _B_PER_W)], rows_v, sem).wait()

    def row_body(i, _):
        v0 = rows_v[i, pl.ds(0 * _L, _L)]
        v1 = rows_v[i, pl.ds(1 * _L, _L)]
        v2 = rows_v[i, pl.ds(2 * _L, _L)]
        v3 = rows_v[i, pl.ds(3 * _L, _L)]
        ss = v0 * v0 + v1 * v1 + v2 * v2 + v3 * v3
        lane = lax.iota(jnp.int32, _L)
        for k in (8, 4, 2, 1):
            ss = ss + _lane_shuffle(ss, lane ^ k)
        s = jnp.maximum(ss, 1e-12)
        inv = _rsqrt16(s)
        rows_v[i, pl.ds(0 * _L, _L)] = v0 * inv
        rows_v[i, pl.ds(1 * _L, _L)] = v1 * inv
        rows_v[i, pl.ds(2 * _L, _L)] = v2 * inv
        rows_v[i, pl.ds(3 * _L, _L)] = v3 * inv
        return 0

    lax.fori_loop(0, _B_PER_W, row_body, 0)
    pltpu.sync_copy(rows_v, out_hbm.at[pl.ds(base, _B_PER_W)])


@jax.jit
def _encode(ids, table):
    mesh = plsc.VectorSubcoreMesh(core_axis_name="c", subcore_axis_name="s")
    call = pl.kernel(
        _sc_body,
        mesh=mesh,
        out_type=jax.ShapeDtypeStruct((BATCH, EMBED_DIM), jnp.float32),
        scratch_types=[
            pltpu.VMEM((_B_PER_W,), jnp.int32),
            pltpu.VMEM((_B_PER_W, EMBED_DIM), jnp.float32),
            pltpu.SemaphoreType.DMA,
        ],
        compiler_params=pltpu.CompilerParams(use_tc_tiling_on_sc=True),
    )
    return call(table, ids.astype(jnp.int32))


def kernel(ids, table):
    return _encode(ids, table)
